# baseline (speedup 1.0000x reference)
import jax
import jax.numpy as jnp
from jax import lax
from jax.experimental import pallas as pl
from jax.experimental.pallas import tpu as pltpu, tpu_sc as plsc

ITEM_NUM = 1000000
ATTR_NUM = 100000
RATING_NUM = 5
EMBED_DIM = 32
ATTR_FNUM = 2
MAX_HIST_LEN = 200
BATCH = 4096
FIELD_NUM = 7

NC = 2
NS = 16
NW = NC * NS
B_PER_W = BATCH // NW
L = MAX_HIST_LEN
LP = 256
APL = 512
INV_L = 1.0 / MAX_HIST_LEN

ITEM_CHUNKS = ((0, 104), (104, 96))
ATTR_CHUNKS = ((0, 104), (104, 104), (208, 104), (312, 88))


def _zeros():
    return jnp.zeros((16,), jnp.float32)


def _worker_base():
    wid = lax.axis_index("s") * NC + lax.axis_index("c")
    return wid, wid * B_PER_W


NP = 1000008
TBC = 512
TNF = NP // TBC
TREM = NP - TNF * TBC
TKMAX = 60


def _transpose_body(src_hbm, out_hbm,
                    in0, in1, out0, out1, rem_in, rem_out,
                    sem_i0, sem_i1, sem_o0, sem_o1):
    st_in = (in0, in1)
    st_out = (out0, out1)
    sem_in = (sem_i0, sem_i1)
    sem_out = (sem_o0, sem_o1)

    wid, _ = _worker_base()
    lane = lax.broadcasted_iota(jnp.int32, (16,), 0)
    lane32 = lane * EMBED_DIM

    def start_in(m, slot):
        c0 = (wid + 32 * m) * TBC
        pltpu.async_copy(src_hbm.at[:, pl.ds(c0, TBC)], st_in[slot],
                         sem_in[slot])

    def wait_in(slot):
        pltpu.make_async_copy(src_hbm.at[:, pl.ds(0, TBC)], st_in[slot],
                              sem_in[slot]).wait()

    def start_out(m, slot):
        e0 = (wid + 32 * m) * TBC * EMBED_DIM
        pltpu.async_copy(st_out[slot], out_hbm.at[pl.ds(e0, TBC * EMBED_DIM)],
                         sem_out[slot])

    def wait_out(slot):
        pltpu.make_async_copy(st_out[slot],
                              out_hbm.at[pl.ds(0, TBC * EMBED_DIM)],
                              sem_out[slot]).wait()

    def transpose(slot):
        @pl.loop(0, TBC // 16)
        def _t(jg):
            j0 = jg * 16
            base_i = j0 * EMBED_DIM
            for d in range(EMBED_DIM):
                v = st_in[slot][d, pl.ds(j0, 16)]
                plsc.store_scatter(st_out[slot], [lane32 + (base_i + d)], v)

    start_in(0, 0)
    start_in(1, 1)
    for m in (0, 1):
        slot = m % 2
        wait_in(slot)
        transpose(slot)
        start_out(m, slot)
        start_in(m + 2, slot)

    @pl.loop(2, TKMAX - 2, step=2)
    def _blocks(k):
        for t in range(2):
            slot = t
            wait_in(slot)
            wait_out(slot)
            transpose(slot)
            start_out(k + t, slot)
            start_in(k + t + 2, slot)

    wait_in(0)
    wait_out(0)
    transpose(0)
    start_out(TKMAX - 2, 0)
    start_in(TKMAX, 0)
    wait_in(1)
    wait_out(1)
    transpose(1)
    start_out(TKMAX - 1, 1)

    @pl.when(wid == 0)
    def _w0_start():
        start_in(TKMAX + 1, 1)

    wait_in(0)
    wait_out(0)
    transpose(0)
    start_out(TKMAX, 0)

    @pl.when(wid == 0)
    def _w0_last():
        wait_in(1)
        wait_out(1)
        transpose(1)
        start_out(TKMAX + 1, 1)

    @pl.when(wid == 1)
    def _remainder():
        pltpu.sync_copy(src_hbm.at[:, pl.ds(TNF * TBC, TREM)], rem_in)

        @pl.loop(0, 5)
        def _tr(jg):
            full = jg < 4
            j0 = jnp.where(full, jg * 16, TREM - 16)
            base_i = j0 * EMBED_DIM
            msk = jnp.where(full, lane >= 0, lane >= 8)
            for d in range(EMBED_DIM):
                v = rem_in[d, pl.ds(j0, 16)]
                plsc.store_scatter(rem_out, [lane32 + (base_i + d)], v,
                                   mask=msk)

        pltpu.sync_copy(rem_out,
                        out_hbm.at[pl.ds(TNF * TBC * EMBED_DIM,
                                         TREM * EMBED_DIM)])

    wait_out(0)
    wait_out(1)


def _attr_body(ha_hbm, hr_hbm, aid_hbm, attr_t, rating_t, out_hbm,
               outbuf, rt_v, av_v,
               ai0, ai1, ri0, ri1, arow0, arow1,
               sem_idx0, sem_idx1, sem_rows0, sem_rows1, sem_a):
    attr_idx = (ai0, ai1)
    rate_idx = (ri0, ri1)
    attr_rows = (arow0, arow1)
    sem_idx = (sem_idx0, sem_idx1)
    sem_rows = (sem_rows0, sem_rows1)

    wid, base = _worker_base()

    pltpu.sync_copy(rating_t, rt_v)

    pltpu.sync_copy(aid_hbm.at[wid], av_v)
    for c in range(2):
        pltpu.async_copy(attr_t.at[av_v.at[c]],
                         arow0.at[pl.ds(c * 128, 128)], sem_a)
    for c in range(2):
        pltpu.make_async_copy(attr_t.at[pl.ds(0, 128)],
                              arow0.at[pl.ds(c * 128, 128)], sem_a).wait()

    @pl.loop(0, B_PER_W)
    def _copy_single(i):
        for v in range(2):
            sl = pl.ds(v * 16, 16)
            outbuf[i, 0, sl] = arow0[2 * i, sl]
            outbuf[i, 1, sl] = arow0[2 * i + 1, sl]

    def start_idx(gb, slot):
        pltpu.async_copy(ha_hbm.at[gb], attr_idx[slot], sem_idx[slot])
        pltpu.async_copy(hr_hbm.at[gb], rate_idx[slot], sem_idx[slot])

    def wait_idx(slot):
        pltpu.make_async_copy(ha_hbm.at[0], attr_idx[slot],
                              sem_idx[slot]).wait()
        pltpu.make_async_copy(hr_hbm.at[0], rate_idx[slot],
                              sem_idx[slot]).wait()

    def start_gathers(slot):
        for off, ln in ATTR_CHUNKS:
            pltpu.async_copy(attr_t.at[attr_idx[slot].at[pl.ds(off, ln)]],
                             attr_rows[slot].at[pl.ds(off, ln)],
                             sem_rows[slot])

    def wait_gathers(slot):
        for off, ln in ATTR_CHUNKS:
            pltpu.make_async_copy(attr_t.at[pl.ds(0, ln)],
                                  attr_rows[slot].at[pl.ds(off, ln)],
                                  sem_rows[slot]).wait()

    def rating(k, slot):
        counts = [jnp.zeros((16,), jnp.int32) for _ in range(RATING_NUM)]
        one = jnp.ones((16,), jnp.int32)
        nil = jnp.zeros((16,), jnp.int32)
        lane = lax.broadcasted_iota(jnp.int32, (16,), 0)
        for i in range(13):
            rv = rate_idx[slot][pl.ds(i * 16, 16)]
            for r in range(RATING_NUM):
                counts[r] = counts[r] + jnp.where(rv == r, one, nil)
        acc = [_zeros(), _zeros()]
        for r in range(RATING_NUM):
            tot = counts[r]
            for sh in (8, 4, 2, 1):
                tot = tot + jnp.take_along_axis(tot, lane ^ sh, axis=0)
            w = tot.astype(jnp.float32) * INV_L
            for v in range(2):
                acc[v] += w * rt_v[r, pl.ds(v * 16, 16)]
        for v in range(2):
            outbuf[k, 4, pl.ds(v * 16, 16)] = acc[v]

    def reduce(k, slot):
        ar = attr_rows[slot]

        def body(l, accs):
            a00, a01, a10, a11 = accs
            s0, s1 = pl.ds(0, 16), pl.ds(16, 16)
            a00 = a00 + ar[2 * l, s0]
            a01 = a01 + ar[2 * l, s1]
            a10 = a10 + ar[2 * l + 1, s0]
            a11 = a11 + ar[2 * l + 1, s1]
            return a00, a01, a10, a11

        init = (_zeros(), _zeros(), _zeros(), _zeros())
        a00, a01, a10, a11 = lax.fori_loop(0, L, body, init, unroll=8)
        s0, s1 = pl.ds(0, 16), pl.ds(16, 16)
        outbuf[k, 2, s0] = a00 * INV_L
        outbuf[k, 2, s1] = a01 * INV_L
        outbuf[k, 3, s0] = a10 * INV_L
        outbuf[k, 3, s1] = a11 * INV_L

    def step(k, slot, do_idx, do_gather):
        wait_gathers(slot)
        rating(k, slot)
        if do_idx:
            start_idx(base + k + 2, slot)
        if do_gather:
            wait_idx(1 - slot)
            start_gathers(1 - slot)
        reduce(k, slot)

    start_idx(base + 0, 0)
    start_idx(base + 1, 1)
    wait_idx(0)
    start_gathers(0)

    @pl.loop(0, B_PER_W - 4, step=2)
    def _main(k):
        step(k, 0, True, True)
        step(k + 1, 1, True, True)

    step(B_PER_W - 4, 0, True, True)
    step(B_PER_W - 3, 1, True, True)
    step(B_PER_W - 2, 0, False, True)
    step(B_PER_W - 1, 1, False, False)

    pltpu.sync_copy(outbuf, out_hbm.at[pl.ds(base, B_PER_W)])


def _item_body(hi_hbm, iid_hbm, item_t, out_hbm,
               outbuf, ii_v,
               ii0, ii1, irow0, irow1,
               sem_idx0, sem_idx1, sem_rows0, sem_rows1, sem_a):
    item_idx = (ii0, ii1)
    item_rows = (irow0, irow1)
    sem_idx = (sem_idx0, sem_idx1)
    sem_rows = (sem_rows0, sem_rows1)

    _, base = _worker_base()

    pltpu.sync_copy(iid_hbm.at[pl.ds(base, B_PER_W)], ii_v)
    pltpu.async_copy(item_t.at[ii_v], irow0.at[pl.ds(0, 128)], sem_a)
    pltpu.make_async_copy(item_t.at[pl.ds(0, 128)],
                          irow0.at[pl.ds(0, 128)], sem_a).wait()

    @pl.loop(0, B_PER_W)
    def _copy_single(i):
        for v in range(2):
            sl = pl.ds(v * 16, 16)
            outbuf[i, 0, sl] = irow0[i, sl]

    def start_idx(gb, slot):
        pltpu.async_copy(hi_hbm.at[gb], item_idx[slot], sem_idx[slot])

    def wait_idx(slot):
        pltpu.make_async_copy(hi_hbm.at[0], item_idx[slot],
                              sem_idx[slot]).wait()

    def start_gathers(slot):
        for off, ln in ITEM_CHUNKS:
            pltpu.async_copy(item_t.at[item_idx[slot].at[pl.ds(off, ln)]],
                             item_rows[slot].at[pl.ds(off, ln)],
                             sem_rows[slot])

    def wait_gathers(slot):
        for off, ln in ITEM_CHUNKS:
            pltpu.make_async_copy(item_t.at[pl.ds(0, ln)],
                                  item_rows[slot].at[pl.ds(off, ln)],
                                  sem_rows[slot]).wait()

    def reduce(k, slot):
        ir = item_rows[slot]

        def body(l, accs):
            i0, i1 = accs
            s0, s1 = pl.ds(0, 16), pl.ds(16, 16)
            return i0 + ir[l, s0], i1 + ir[l, s1]

        i0, i1 = lax.fori_loop(0, L, body, (_zeros(), _zeros()), unroll=8)
        s0, s1 = pl.ds(0, 16), pl.ds(16, 16)
        outbuf[k, 1, s0] = i0 * INV_L
        outbuf[k, 1, s1] = i1 * INV_L

    def step(k, slot, do_idx, do_gather):
        wait_gathers(slot)
        if do_idx:
            start_idx(base + k + 2, slot)
        if do_gather:
            wait_idx(1 - slot)
            start_gathers(1 - slot)
        reduce(k, slot)

    start_idx(base + 0, 0)
    start_idx(base + 1, 1)
    wait_idx(0)
    start_gathers(0)

    @pl.loop(0, B_PER_W - 4, step=2)
    def _main(k):
        step(k, 0, True, True)
        step(k + 1, 1, True, True)

    step(B_PER_W - 4, 0, True, True)
    step(B_PER_W - 3, 1, True, True)
    step(B_PER_W - 2, 0, False, True)
    step(B_PER_W - 1, 1, False, False)

    pltpu.sync_copy(outbuf, out_hbm.at[pl.ds(base, B_PER_W)])


@jax.jit
def _run(hi_p, ha_p, hr_p, iid_a, aid3, item_tableT, attr_table,
         rating_table):
    mesh = plsc.VectorSubcoreMesh(core_axis_name="c", subcore_axis_name="s")
    params = pltpu.CompilerParams(use_tc_tiling_on_sc=False)
    tparams = pltpu.CompilerParams(use_tc_tiling_on_sc=False,
                                   needs_layout_passes=False)
    trans_f = pl.kernel(
        _transpose_body,
        out_type=jax.ShapeDtypeStruct((NP * EMBED_DIM,), jnp.float32),
        mesh=mesh,
        scratch_types=[
            pltpu.VMEM((EMBED_DIM, TBC), jnp.float32),
            pltpu.VMEM((EMBED_DIM, TBC), jnp.float32),
            pltpu.VMEM((TBC * EMBED_DIM,), jnp.float32),
            pltpu.VMEM((TBC * EMBED_DIM,), jnp.float32),
            pltpu.VMEM((EMBED_DIM, TREM), jnp.float32),
            pltpu.VMEM((TREM * EMBED_DIM,), jnp.float32),
            pltpu.SemaphoreType.DMA,
            pltpu.SemaphoreType.DMA,
            pltpu.SemaphoreType.DMA,
            pltpu.SemaphoreType.DMA,
        ],
        compiler_params=tparams,
    )
    attr_f = pl.kernel(
        _attr_body,
        out_type=jax.ShapeDtypeStruct((BATCH, 5, EMBED_DIM), jnp.float32),
        mesh=mesh,
        scratch_types=[
            pltpu.VMEM((B_PER_W, 5, EMBED_DIM), jnp.float32),
            pltpu.VMEM((RATING_NUM + 1, EMBED_DIM), jnp.float32),
            pltpu.VMEM((2, 128), jnp.int32),
            pltpu.VMEM((APL,), jnp.int32),
            pltpu.VMEM((APL,), jnp.int32),
            pltpu.VMEM((LP,), jnp.int32),
            pltpu.VMEM((LP,), jnp.int32),
            pltpu.VMEM((APL, EMBED_DIM), jnp.float32),
            pltpu.VMEM((APL, EMBED_DIM), jnp.float32),
            pltpu.SemaphoreType.DMA,
            pltpu.SemaphoreType.DMA,
            pltpu.SemaphoreType.DMA,
            pltpu.SemaphoreType.DMA,
            pltpu.SemaphoreType.DMA,
        ],
        compiler_params=params,
    )
    item_f = pl.kernel(
        _item_body,
        out_type=jax.ShapeDtypeStruct((BATCH, 2, EMBED_DIM), jnp.float32),
        mesh=mesh,
        scratch_types=[
            pltpu.VMEM((B_PER_W, 2, EMBED_DIM), jnp.float32),
            pltpu.VMEM((B_PER_W,), jnp.int32),
            pltpu.VMEM((LP,), jnp.int32),
            pltpu.VMEM((LP,), jnp.int32),
            pltpu.VMEM((LP, EMBED_DIM), jnp.float32),
            pltpu.VMEM((LP, EMBED_DIM), jnp.float32),
            pltpu.SemaphoreType.DMA,
            pltpu.SemaphoreType.DMA,
            pltpu.SemaphoreType.DMA,
            pltpu.SemaphoreType.DMA,
            pltpu.SemaphoreType.DMA,
        ],
        compiler_params=params,
    )
    out_a = attr_f(ha_p, hr_p, aid3, attr_table, rating_table)
    item_lin = trans_f(item_tableT).reshape(NP, EMBED_DIM)
    out_i = item_f(hi_p, iid_a, item_lin)
    return jnp.concatenate(
        [out_i[:, 0:1], out_a[:, 0:2], out_i[:, 1:2], out_a[:, 2:5]], axis=1)


def kernel(hist_iid_seq, hist_aid_seq, hist_rate_seq, hist_seq_len, iid, aid,
           lb, item_table, attr_table, rating_table):
    del hist_seq_len, lb
    hi_p = jnp.pad(hist_iid_seq.astype(jnp.int32), ((0, 0), (0, LP - L)))
    ha = hist_aid_seq.astype(jnp.int32).reshape(BATCH, 2 * L)
    ha_p = jnp.pad(ha, ((0, 0), (0, APL - 2 * L)))
    hr_p = jnp.pad(hist_rate_seq.astype(jnp.int32), ((0, 0), (0, LP - L)),
                   constant_values=RATING_NUM)
    aid3 = aid.astype(jnp.int32).reshape(NW, 2, B_PER_W)
    item_tT = jnp.pad(jnp.transpose(item_table.astype(jnp.float32), (1, 0)),
                      ((0, 0), (0, NP - (ITEM_NUM + 1))))
    return _run(hi_p, ha_p, hr_p, iid.astype(jnp.int32), aid3,
                item_tT,
                attr_table.astype(jnp.float32),
                rating_table.astype(jnp.float32))


# speedup vs baseline: 4.9602x; 4.9602x over previous
import jax
import jax.numpy as jnp
from jax import lax
from jax.experimental import pallas as pl
from jax.experimental.pallas import tpu as pltpu, tpu_sc as plsc

ITEM_NUM = 1000000
ATTR_NUM = 100000
RATING_NUM = 5
EMBED_DIM = 32
ATTR_FNUM = 2
MAX_HIST_LEN = 200
BATCH = 4096
FIELD_NUM = 7

NC = 2
NS = 16
NW = NC * NS
B_PER_W = BATCH // NW
L = MAX_HIST_LEN
LP = 256
APL = 512
INV_L = 1.0 / MAX_HIST_LEN

ITEM_CHUNKS = ((0, 104), (104, 96))
ATTR_CHUNKS = ((0, 104), (104, 104), (208, 104), (312, 88))


def _zeros():
    return jnp.zeros((16,), jnp.float32)


def _worker_base():
    wid = lax.axis_index("s") * NC + lax.axis_index("c")
    return wid, wid * B_PER_W


def _attr_body(ha_hbm, hr_hbm, aid_hbm, attr_t, rating_t, out_hbm,
               outbuf, rt_v, av_v,
               ai0, ai1, ri0, ri1, arow0, arow1,
               sem_idx0, sem_idx1, sem_rows0, sem_rows1, sem_a):
    attr_idx = (ai0, ai1)
    rate_idx = (ri0, ri1)
    attr_rows = (arow0, arow1)
    sem_idx = (sem_idx0, sem_idx1)
    sem_rows = (sem_rows0, sem_rows1)

    wid, base = _worker_base()

    pltpu.sync_copy(rating_t, rt_v)

    pltpu.sync_copy(aid_hbm.at[wid], av_v)
    for c in range(2):
        pltpu.async_copy(attr_t.at[av_v.at[c]],
                         arow0.at[pl.ds(c * 128, 128)], sem_a)
    for c in range(2):
        pltpu.make_async_copy(attr_t.at[pl.ds(0, 128)],
                              arow0.at[pl.ds(c * 128, 128)], sem_a).wait()

    @pl.loop(0, B_PER_W)
    def _copy_single(i):
        for v in range(2):
            sl = pl.ds(v * 16, 16)
            outbuf[i, 0, sl] = arow0[2 * i, sl]
            outbuf[i, 1, sl] = arow0[2 * i + 1, sl]

    def start_idx(gb, slot):
        pltpu.async_copy(ha_hbm.at[gb], attr_idx[slot], sem_idx[slot])
        pltpu.async_copy(hr_hbm.at[gb], rate_idx[slot], sem_idx[slot])

    def wait_idx(slot):
        pltpu.make_async_copy(ha_hbm.at[0], attr_idx[slot],
                              sem_idx[slot]).wait()
        pltpu.make_async_copy(hr_hbm.at[0], rate_idx[slot],
                              sem_idx[slot]).wait()

    def start_gathers(slot):
        for off, ln in ATTR_CHUNKS:
            pltpu.async_copy(attr_t.at[attr_idx[slot].at[pl.ds(off, ln)]],
                             attr_rows[slot].at[pl.ds(off, ln)],
                             sem_rows[slot])

    def wait_gathers(slot):
        for off, ln in ATTR_CHUNKS:
            pltpu.make_async_copy(attr_t.at[pl.ds(0, ln)],
                                  attr_rows[slot].at[pl.ds(off, ln)],
                                  sem_rows[slot]).wait()

    def rating(k, slot):
        counts = [jnp.zeros((16,), jnp.int32) for _ in range(RATING_NUM)]
        one = jnp.ones((16,), jnp.int32)
        nil = jnp.zeros((16,), jnp.int32)
        lane = lax.broadcasted_iota(jnp.int32, (16,), 0)
        for i in range(13):
            rv = rate_idx[slot][pl.ds(i * 16, 16)]
            for r in range(RATING_NUM):
                counts[r] = counts[r] + jnp.where(rv == r, one, nil)
        acc = [_zeros(), _zeros()]
        for r in range(RATING_NUM):
            tot = counts[r]
            for sh in (8, 4, 2, 1):
                tot = tot + jnp.take_along_axis(tot, lane ^ sh, axis=0)
            w = tot.astype(jnp.float32) * INV_L
            for v in range(2):
                acc[v] += w * rt_v[r, pl.ds(v * 16, 16)]
        for v in range(2):
            outbuf[k, 4, pl.ds(v * 16, 16)] = acc[v]

    def reduce(k, slot):
        ar = attr_rows[slot]

        def body(l, accs):
            a00, a01, a10, a11 = accs
            s0, s1 = pl.ds(0, 16), pl.ds(16, 16)
            a00 = a00 + ar[2 * l, s0]
            a01 = a01 + ar[2 * l, s1]
            a10 = a10 + ar[2 * l + 1, s0]
            a11 = a11 + ar[2 * l + 1, s1]
            return a00, a01, a10, a11

        init = (_zeros(), _zeros(), _zeros(), _zeros())
        a00, a01, a10, a11 = lax.fori_loop(0, L, body, init, unroll=8)
        s0, s1 = pl.ds(0, 16), pl.ds(16, 16)
        outbuf[k, 2, s0] = a00 * INV_L
        outbuf[k, 2, s1] = a01 * INV_L
        outbuf[k, 3, s0] = a10 * INV_L
        outbuf[k, 3, s1] = a11 * INV_L

    def step(k, slot, do_idx, do_gather):
        wait_gathers(slot)
        rating(k, slot)
        if do_idx:
            start_idx(base + k + 2, slot)
        if do_gather:
            wait_idx(1 - slot)
            start_gathers(1 - slot)
        reduce(k, slot)

    start_idx(base + 0, 0)
    start_idx(base + 1, 1)
    wait_idx(0)
    start_gathers(0)

    @pl.loop(0, B_PER_W - 4, step=2)
    def _main(k):
        step(k, 0, True, True)
        step(k + 1, 1, True, True)

    step(B_PER_W - 4, 0, True, True)
    step(B_PER_W - 3, 1, True, True)
    step(B_PER_W - 2, 0, False, True)
    step(B_PER_W - 1, 1, False, False)

    pltpu.sync_copy(outbuf, out_hbm.at[pl.ds(base, B_PER_W)])


def _item_body(hi_hbm, iid_hbm, item_t, out_hbm,
               outbuf, ii_v,
               ii0, ii1, irow0, irow1,
               sem_idx0, sem_idx1, sem_rows0, sem_rows1, sem_a):
    item_idx = (ii0, ii1)
    item_rows = (irow0, irow1)
    sem_idx = (sem_idx0, sem_idx1)
    sem_rows = (sem_rows0, sem_rows1)

    _, base = _worker_base()

    pltpu.sync_copy(iid_hbm.at[pl.ds(base, B_PER_W)], ii_v)
    pltpu.async_copy(item_t.at[ii_v], irow0.at[pl.ds(0, 128)], sem_a)
    pltpu.make_async_copy(item_t.at[pl.ds(0, 128)],
                          irow0.at[pl.ds(0, 128)], sem_a).wait()

    @pl.loop(0, B_PER_W)
    def _copy_single(i):
        for v in range(2):
            sl = pl.ds(v * 16, 16)
            outbuf[i, 0, sl] = irow0[i, sl]

    def start_idx(gb, slot):
        pltpu.async_copy(hi_hbm.at[gb], item_idx[slot], sem_idx[slot])

    def wait_idx(slot):
        pltpu.make_async_copy(hi_hbm.at[0], item_idx[slot],
                              sem_idx[slot]).wait()

    def start_gathers(slot):
        for off, ln in ITEM_CHUNKS:
            pltpu.async_copy(item_t.at[item_idx[slot].at[pl.ds(off, ln)]],
                             item_rows[slot].at[pl.ds(off, ln)],
                             sem_rows[slot])

    def wait_gathers(slot):
        for off, ln in ITEM_CHUNKS:
            pltpu.make_async_copy(item_t.at[pl.ds(0, ln)],
                                  item_rows[slot].at[pl.ds(off, ln)],
                                  sem_rows[slot]).wait()

    def reduce(k, slot):
        ir = item_rows[slot]

        def body(l, accs):
            i0, i1 = accs
            s0, s1 = pl.ds(0, 16), pl.ds(16, 16)
            return i0 + ir[l, s0], i1 + ir[l, s1]

        i0, i1 = lax.fori_loop(0, L, body, (_zeros(), _zeros()), unroll=8)
        s0, s1 = pl.ds(0, 16), pl.ds(16, 16)
        outbuf[k, 1, s0] = i0 * INV_L
        outbuf[k, 1, s1] = i1 * INV_L

    def step(k, slot, do_idx, do_gather):
        wait_gathers(slot)
        if do_idx:
            start_idx(base + k + 2, slot)
        if do_gather:
            wait_idx(1 - slot)
            start_gathers(1 - slot)
        reduce(k, slot)

    start_idx(base + 0, 0)
    start_idx(base + 1, 1)
    wait_idx(0)
    start_gathers(0)

    @pl.loop(0, B_PER_W - 4, step=2)
    def _main(k):
        step(k, 0, True, True)
        step(k + 1, 1, True, True)

    step(B_PER_W - 4, 0, True, True)
    step(B_PER_W - 3, 1, True, True)
    step(B_PER_W - 2, 0, False, True)
    step(B_PER_W - 1, 1, False, False)

    pltpu.sync_copy(outbuf, out_hbm.at[pl.ds(base, B_PER_W)])


@jax.jit
def _run(hi_p, ha_p, hr_p, iid_a, aid3, item_table, attr_table,
         rating_table):
    mesh = plsc.VectorSubcoreMesh(core_axis_name="c", subcore_axis_name="s")
    params = pltpu.CompilerParams(use_tc_tiling_on_sc=False)
    attr_f = pl.kernel(
        _attr_body,
        out_type=jax.ShapeDtypeStruct((BATCH, 5, EMBED_DIM), jnp.float32),
        mesh=mesh,
        scratch_types=[
            pltpu.VMEM((B_PER_W, 5, EMBED_DIM), jnp.float32),
            pltpu.VMEM((RATING_NUM + 1, EMBED_DIM), jnp.float32),
            pltpu.VMEM((2, 128), jnp.int32),
            pltpu.VMEM((APL,), jnp.int32),
            pltpu.VMEM((APL,), jnp.int32),
            pltpu.VMEM((LP,), jnp.int32),
            pltpu.VMEM((LP,), jnp.int32),
            pltpu.VMEM((APL, EMBED_DIM), jnp.float32),
            pltpu.VMEM((APL, EMBED_DIM), jnp.float32),
            pltpu.SemaphoreType.DMA,
            pltpu.SemaphoreType.DMA,
            pltpu.SemaphoreType.DMA,
            pltpu.SemaphoreType.DMA,
            pltpu.SemaphoreType.DMA,
        ],
        compiler_params=params,
    )
    item_f = pl.kernel(
        _item_body,
        out_type=jax.ShapeDtypeStruct((BATCH, 2, EMBED_DIM), jnp.float32),
        mesh=mesh,
        scratch_types=[
            pltpu.VMEM((B_PER_W, 2, EMBED_DIM), jnp.float32),
            pltpu.VMEM((B_PER_W,), jnp.int32),
            pltpu.VMEM((LP,), jnp.int32),
            pltpu.VMEM((LP,), jnp.int32),
            pltpu.VMEM((LP, EMBED_DIM), jnp.float32),
            pltpu.VMEM((LP, EMBED_DIM), jnp.float32),
            pltpu.SemaphoreType.DMA,
            pltpu.SemaphoreType.DMA,
            pltpu.SemaphoreType.DMA,
            pltpu.SemaphoreType.DMA,
            pltpu.SemaphoreType.DMA,
        ],
        compiler_params=params,
    )
    out_a = attr_f(ha_p, hr_p, aid3, attr_table, rating_table)
    out_i = item_f(hi_p, iid_a, item_table)
    return jnp.concatenate(
        [out_i[:, 0:1], out_a[:, 0:2], out_i[:, 1:2], out_a[:, 2:5]], axis=1)


def kernel(hist_iid_seq, hist_aid_seq, hist_rate_seq, hist_seq_len, iid, aid,
           lb, item_table, attr_table, rating_table):
    del hist_seq_len, lb
    hi_p = jnp.pad(hist_iid_seq.astype(jnp.int32), ((0, 0), (0, LP - L)))
    ha = hist_aid_seq.astype(jnp.int32).reshape(BATCH, 2 * L)
    ha_p = jnp.pad(ha, ((0, 0), (0, APL - 2 * L)))
    hr_p = jnp.pad(hist_rate_seq.astype(jnp.int32), ((0, 0), (0, LP - L)),
                   constant_values=RATING_NUM)
    aid3 = aid.astype(jnp.int32).reshape(NW, 2, B_PER_W)
    return _run(hi_p, ha_p, hr_p, iid.astype(jnp.int32), aid3,
                item_table.astype(jnp.float32),
                attr_table.astype(jnp.float32),
                rating_table.astype(jnp.float32))
